# baseline (speedup 1.0000x reference)
import functools

import jax
import jax.numpy as jnp
from jax import lax
from jax.experimental import pallas as pl
from jax.experimental.pallas import tpu as pltpu
from jax.experimental.pallas import tpu_sc as plsc

NC = 2
NS = 16
C = 96


def _segment_sum_sc(h, src2d, dst3d, zeros):
  n, d = h.shape
  nw, ept = src2d.shape
  _, kpt, c = dst3d.shape
  npad = ((n + 8 * NS - 1) // (8 * NS)) * (8 * NS)
  rpt = npad // NS
  mesh = plsc.VectorSubcoreMesh(core_axis_name="c", subcore_axis_name="s")

  @functools.partial(
      pl.kernel,
      out_type=jax.ShapeDtypeStruct((NC * npad, d), jnp.float32),
      mesh=mesh,
      scratch_types=[
          pltpu.VMEM_SHARED((npad, d), jnp.float32),
          pltpu.VMEM((ept,), jnp.int32),
          pltpu.VMEM((kpt, c), jnp.int32),
          pltpu.VMEM((c, d), jnp.float32),
          pltpu.VMEM((c, d), jnp.float32),
          pltpu.SemaphoreType.DMA,
          pltpu.SemaphoreType.DMA,
      ],
  )
  def seg_sum(h_hbm, src_hbm, dst_hbm, zeros_hbm, out_hbm, acc,
              src_idx, dst_idx, rows0, rows1, sem0, sem1):
    cid = lax.axis_index("c")
    sid = lax.axis_index("s")
    wid = cid * NS + sid
    pltpu.sync_copy(src_hbm.at[wid], src_idx)
    pltpu.sync_copy(dst_hbm.at[wid], dst_idx)
    rz = sid * rpt
    pltpu.sync_copy(zeros_hbm.at[pl.ds(rz, rpt)], acc.at[pl.ds(rz, rpt)])
    plsc.subcore_barrier()

    pltpu.async_copy(h_hbm.at[src_idx.at[pl.ds(0, c)]], rows0, sem0)
    npairs = kpt // 2

    def pair(j, carry):
      k = j * 2
      pltpu.make_async_copy(h_hbm.at[src_idx.at[pl.ds(k * c, c)]], rows0,
                            sem0).wait()
      d1 = pltpu.async_copy(h_hbm.at[src_idx.at[pl.ds((k + 1) * c, c)]],
                            rows1, sem1)
      pltpu.sync_copy(rows0, acc.at[dst_idx.at[k]], add=True)
      pltpu.async_copy(h_hbm.at[src_idx.at[pl.ds((k + 2) * c, c)]], rows0,
                       sem0)
      d1.wait()
      pltpu.sync_copy(rows1, acc.at[dst_idx.at[k + 1]], add=True)
      return carry

    lax.fori_loop(0, npairs, pair, 0)
    pltpu.make_async_copy(h_hbm.at[src_idx.at[pl.ds((kpt - 1) * c, c)]],
                          rows0, sem0).wait()
    pltpu.sync_copy(rows0, acc.at[dst_idx.at[kpt - 1]], add=True)
    plsc.subcore_barrier()
    pltpu.sync_copy(acc.at[pl.ds(rz, rpt)],
                    out_hbm.at[pl.ds(cid * npad + rz, rpt)])

  return seg_sum(h, src2d, dst3d, zeros), npad


def _mm_kernel(x_ref, w_ref, o_ref):
  o_ref[...] = jnp.dot(x_ref[...], w_ref[...],
                       preferred_element_type=jnp.float32,
                       precision=lax.Precision.HIGHEST)


def _relu_mm_kernel(n, p_ref, w_ref, o_ref):
  h = jnp.maximum(p_ref[0, :n] + p_ref[1, :n], 0.0)
  o_ref[...] = jnp.dot(h, w_ref[...],
                       preferred_element_type=jnp.float32,
                       precision=lax.Precision.HIGHEST)


def _add_log_softmax_kernel(n, p_ref, o_ref):
  h = p_ref[0, :n] + p_ref[1, :n]
  m = jnp.max(h, axis=1, keepdims=True)
  lse = jnp.log(jnp.sum(jnp.exp(h - m), axis=1, keepdims=True)) + m
  o_ref[...] = h - lse


def kernel(x, edge_index, W1, W2):
  n, d = x.shape
  e = edge_index.shape[1]
  nw = NC * NS
  ept = e // nw
  kpt = -(-ept // C)
  npad = ((n + 8 * NS - 1) // (8 * NS)) * (8 * NS)
  pad = kpt * C - ept
  dst2 = edge_index[0].reshape(nw, ept)
  src2 = edge_index[1].reshape(nw, ept)
  if pad:
    dst2 = jnp.pad(dst2, ((0, 0), (0, pad)), constant_values=npad - 1)
    src2 = jnp.pad(src2, ((0, 0), (0, pad)))
  dst3d = dst2.reshape(nw, kpt, C)
  src2d = src2.reshape(nw, kpt * C)
  zeros = jnp.zeros((npad, d), jnp.float32)

  h1 = pl.pallas_call(
      _mm_kernel,
      out_shape=jax.ShapeDtypeStruct((n, d), jnp.float32),
  )(x, W1)
  p1, _ = _segment_sum_sc(h1, src2d, dst3d, zeros)
  h2 = pl.pallas_call(
      functools.partial(_relu_mm_kernel, n),
      out_shape=jax.ShapeDtypeStruct((n, d), jnp.float32),
  )(p1.reshape(NC, npad, d), W2)
  p2, _ = _segment_sum_sc(h2, src2d, dst3d, zeros)
  return pl.pallas_call(
      functools.partial(_add_log_softmax_kernel, n),
      out_shape=jax.ShapeDtypeStruct((n, d), jnp.float32),
  )(p2.reshape(NC, npad, d))


# speedup vs baseline: 1.0009x; 1.0009x over previous
import functools

import jax
import jax.numpy as jnp
from jax import lax
from jax.experimental import pallas as pl
from jax.experimental.pallas import tpu as pltpu
from jax.experimental.pallas import tpu_sc as plsc

NC = 2
NS = 16
C = 96


def _segment_sum_sc(h, src2d, dst3d, zeros):
  n, d = h.shape
  nw, ept = src2d.shape
  _, kpt, c = dst3d.shape
  npad = ((n + 8 * NS - 1) // (8 * NS)) * (8 * NS)
  rpt = npad // NS
  mesh = plsc.VectorSubcoreMesh(core_axis_name="c", subcore_axis_name="s")

  @functools.partial(
      pl.kernel,
      out_type=jax.ShapeDtypeStruct((NC * npad, d), jnp.float32),
      mesh=mesh,
      scratch_types=[
          pltpu.VMEM_SHARED((npad, d), jnp.float32),
          pltpu.VMEM((ept,), jnp.int32),
          pltpu.VMEM((kpt, c), jnp.int32),
          pltpu.VMEM((c, d), jnp.float32),
          pltpu.VMEM((c, d), jnp.float32),
          pltpu.SemaphoreType.DMA,
          pltpu.SemaphoreType.DMA,
      ],
  )
  def seg_sum(h_hbm, src_hbm, dst_hbm, zeros_hbm, out_hbm, acc,
              src_idx, dst_idx, rows0, rows1, sem0, sem1):
    cid = lax.axis_index("c")
    sid = lax.axis_index("s")
    wid = cid * NS + sid
    pltpu.sync_copy(src_hbm.at[wid], src_idx)
    pltpu.sync_copy(dst_hbm.at[wid], dst_idx)
    rz = sid * rpt
    pltpu.sync_copy(zeros_hbm.at[pl.ds(rz, rpt)], acc.at[pl.ds(rz, rpt)])
    plsc.subcore_barrier()

    pltpu.async_copy(h_hbm.at[src_idx.at[pl.ds(0, c)]], rows0, sem0)
    npairs = kpt // 2

    def pair(j, carry):
      k = j * 2
      pltpu.make_async_copy(h_hbm.at[src_idx.at[pl.ds(k * c, c)]], rows0,
                            sem0).wait()
      d1 = pltpu.async_copy(h_hbm.at[src_idx.at[pl.ds((k + 1) * c, c)]],
                            rows1, sem1)
      pltpu.sync_copy(rows0, acc.at[dst_idx.at[k]], add=True)
      pltpu.async_copy(h_hbm.at[src_idx.at[pl.ds((k + 2) * c, c)]], rows0,
                       sem0)
      d1.wait()
      pltpu.sync_copy(rows1, acc.at[dst_idx.at[k + 1]], add=True)
      return carry

    lax.fori_loop(0, npairs, pair, 0)
    pltpu.make_async_copy(h_hbm.at[src_idx.at[pl.ds((kpt - 1) * c, c)]],
                          rows0, sem0).wait()
    pltpu.sync_copy(rows0, acc.at[dst_idx.at[kpt - 1]], add=True)
    plsc.subcore_barrier()
    pltpu.sync_copy(acc.at[pl.ds(rz, rpt)],
                    out_hbm.at[pl.ds(cid * npad + rz, rpt)])

  return seg_sum(h, src2d, dst3d, zeros), npad


def _mm_kernel(x_ref, w_ref, o_ref):
  o_ref[...] = jnp.dot(x_ref[...], w_ref[...],
                       preferred_element_type=jnp.float32,
                       precision=lax.Precision.HIGHEST)


def _relu_mm_kernel(n, p_ref, w_ref, o_ref):
  h = jnp.maximum(p_ref[0, :n] + p_ref[1, :n], 0.0)
  o_ref[...] = jnp.dot(h, w_ref[...],
                       preferred_element_type=jnp.float32,
                       precision=lax.Precision.HIGHEST)


def _add_log_softmax_kernel(n, p_ref, o_ref):
  h = p_ref[0, :n] + p_ref[1, :n]
  m = jnp.max(h, axis=1, keepdims=True)
  lse = jnp.log(jnp.sum(jnp.exp(h - m), axis=1, keepdims=True)) + m
  o_ref[...] = h - lse


def kernel(x, edge_index, W1, W2):
  n, d = x.shape
  e = edge_index.shape[1]
  nw = NC * NS
  ept = e // nw
  kpt = -(-ept // C)
  npad = ((n + 8 * NS - 1) // (8 * NS)) * (8 * NS)
  pad = kpt * C - ept
  dst2 = edge_index[0].reshape(nw, ept)
  src2 = edge_index[1].reshape(nw, ept)
  if pad:
    pad_dst = (jnp.arange(nw)[:, None] * 7 + jnp.arange(pad)[None, :]
               ) % (npad - n) + n
    dst2 = jnp.concatenate([dst2, pad_dst.astype(jnp.int32)], axis=1)
    src2 = jnp.pad(src2, ((0, 0), (0, pad)))
  dst3d = dst2.reshape(nw, kpt, C)
  src2d = src2.reshape(nw, kpt * C)
  zeros = jnp.zeros((npad, d), jnp.float32)

  h1 = pl.pallas_call(
      _mm_kernel,
      out_shape=jax.ShapeDtypeStruct((n, d), jnp.float32),
  )(x, W1)
  p1, _ = _segment_sum_sc(h1, src2d, dst3d, zeros)
  h2 = pl.pallas_call(
      functools.partial(_relu_mm_kernel, n),
      out_shape=jax.ShapeDtypeStruct((n, d), jnp.float32),
  )(p1.reshape(NC, npad, d), W2)
  p2, _ = _segment_sum_sc(h2, src2d, dst3d, zeros)
  return pl.pallas_call(
      functools.partial(_add_log_softmax_kernel, n),
      out_shape=jax.ShapeDtypeStruct((n, d), jnp.float32),
  )(p2.reshape(NC, npad, d))


# speedup vs baseline: 1.6432x; 1.6418x over previous
import functools

import jax
import jax.numpy as jnp
from jax import lax
from jax.experimental import pallas as pl
from jax.experimental.pallas import tpu as pltpu
from jax.experimental.pallas import tpu_sc as plsc

NC = 2
NS = 16
C = 125
G = 16


def _segment_sum_sc(h, src3d, dst3d, zeros):
  n, d = h.shape
  nw, kpt, c = src3d.shape
  npad = ((n + 8 * NS - 1) // (8 * NS)) * (8 * NS)
  rpt = npad // NS
  mesh = plsc.VectorSubcoreMesh(core_axis_name="c", subcore_axis_name="s")

  @functools.partial(
      pl.kernel,
      out_type=jax.ShapeDtypeStruct((NC * npad, d), jnp.float32),
      mesh=mesh,
      scratch_types=[
          pltpu.VMEM_SHARED((npad, d), jnp.float32),
          pltpu.VMEM((G, c), jnp.int32),
          pltpu.VMEM((G, c), jnp.int32),
          pltpu.VMEM((G, c), jnp.int32),
          pltpu.VMEM((G, c), jnp.int32),
          pltpu.VMEM((c, d), jnp.float32),
          pltpu.VMEM((c, d), jnp.float32),
          pltpu.SemaphoreType.DMA,
          pltpu.SemaphoreType.DMA,
          pltpu.SemaphoreType.DMA,
          pltpu.SemaphoreType.DMA,
      ],
  )
  def seg_sum(h_hbm, src_hbm, dst_hbm, zeros_hbm, out_hbm, acc,
              srcg0, dstg0, srcg1, dstg1, rows0, rows1,
              sem0, sem1, semi0, semi1):
    cid = lax.axis_index("c")
    sid = lax.axis_index("s")
    wid = cid * NS + sid
    ngroups = kpt // G
    npairs = G // 2
    srcg = (srcg0, srcg1)
    dstg = (dstg0, dstg1)
    semi = (semi0, semi1)

    def prefetch_idx(g):
      pltpu.async_copy(src_hbm.at[wid, pl.ds(g * G, G)], srcg[g % 2],
                       semi[g % 2])
      pltpu.async_copy(dst_hbm.at[wid, pl.ds(g * G, G)], dstg[g % 2],
                       semi[g % 2])

    def wait_idx(g):
      pltpu.make_async_copy(src_hbm.at[wid, pl.ds(g * G, G)], srcg[g % 2],
                            semi[g % 2]).wait()
      pltpu.make_async_copy(dst_hbm.at[wid, pl.ds(g * G, G)], dstg[g % 2],
                            semi[g % 2]).wait()

    prefetch_idx(0)
    if ngroups > 1:
      prefetch_idx(1)
    rz = sid * rpt
    pltpu.sync_copy(zeros_hbm.at[pl.ds(rz, rpt)], acc.at[pl.ds(rz, rpt)])
    plsc.subcore_barrier()
    wait_idx(0)
    pltpu.async_copy(h_hbm.at[srcg0.at[0]], rows0, sem0)

    for g in range(ngroups):
      sg, dg = srcg[g % 2], dstg[g % 2]
      if g + 1 < ngroups:
        wait_idx(g + 1)

      def pair(j, carry, sg=sg, dg=dg):
        k = j * 2
        pltpu.make_async_copy(h_hbm.at[sg.at[k]], rows0, sem0).wait()
        d1 = pltpu.async_copy(h_hbm.at[sg.at[k + 1]], rows1, sem1)
        pltpu.sync_copy(rows0, acc.at[dg.at[k]], add=True)
        pltpu.async_copy(h_hbm.at[sg.at[k + 2]], rows0, sem0)
        d1.wait()
        pltpu.sync_copy(rows1, acc.at[dg.at[k + 1]], add=True)
        return carry

      lax.fori_loop(0, npairs - 1, pair, 0)
      k = G - 2
      pltpu.make_async_copy(h_hbm.at[sg.at[k]], rows0, sem0).wait()
      d1 = pltpu.async_copy(h_hbm.at[sg.at[k + 1]], rows1, sem1)
      pltpu.sync_copy(rows0, acc.at[dg.at[k]], add=True)
      if g + 1 < ngroups:
        pltpu.async_copy(h_hbm.at[srcg[(g + 1) % 2].at[0]], rows0, sem0)
      d1.wait()
      pltpu.sync_copy(rows1, acc.at[dg.at[k + 1]], add=True)
      if g + 2 < ngroups:
        prefetch_idx(g + 2)
    plsc.subcore_barrier()
    pltpu.sync_copy(acc.at[pl.ds(rz, rpt)],
                    out_hbm.at[pl.ds(cid * npad + rz, rpt)])

  return seg_sum(h, src3d, dst3d, zeros), npad


def _mm_kernel(x_ref, w_ref, o_ref):
  o_ref[...] = jnp.dot(x_ref[...], w_ref[...],
                       preferred_element_type=jnp.float32,
                       precision=lax.Precision.HIGHEST)


def _relu_mm_kernel(n, p_ref, w_ref, o_ref):
  h = jnp.maximum(p_ref[0, :n] + p_ref[1, :n], 0.0)
  o_ref[...] = jnp.dot(h, w_ref[...],
                       preferred_element_type=jnp.float32,
                       precision=lax.Precision.HIGHEST)


def _add_log_softmax_kernel(n, p_ref, o_ref):
  h = p_ref[0, :n] + p_ref[1, :n]
  m = jnp.max(h, axis=1, keepdims=True)
  lse = jnp.log(jnp.sum(jnp.exp(h - m), axis=1, keepdims=True)) + m
  o_ref[...] = h - lse


def kernel(x, edge_index, W1, W2):
  n, d = x.shape
  e = edge_index.shape[1]
  nw = NC * NS
  kpt = e // (C * nw)
  npad = ((n + 8 * NS - 1) // (8 * NS)) * (8 * NS)
  dst3d = edge_index[0].reshape(nw, kpt, C)
  src3d = edge_index[1].reshape(nw, kpt, C)
  zeros = jnp.zeros((npad, d), jnp.float32)

  h1 = pl.pallas_call(
      _mm_kernel,
      out_shape=jax.ShapeDtypeStruct((n, d), jnp.float32),
  )(x, W1)
  p1, _ = _segment_sum_sc(h1, src3d, dst3d, zeros)
  h2 = pl.pallas_call(
      functools.partial(_relu_mm_kernel, n),
      out_shape=jax.ShapeDtypeStruct((n, d), jnp.float32),
  )(p1.reshape(NC, npad, d), W2)
  p2, _ = _segment_sum_sc(h2, src3d, dst3d, zeros)
  return pl.pallas_call(
      functools.partial(_add_log_softmax_kernel, n),
      out_shape=jax.ShapeDtypeStruct((n, d), jnp.float32),
  )(p2.reshape(NC, npad, d))


# speedup vs baseline: 1.7424x; 1.0603x over previous
import functools

import jax
import jax.numpy as jnp
from jax import lax
from jax.experimental import pallas as pl
from jax.experimental.pallas import tpu as pltpu
from jax.experimental.pallas import tpu_sc as plsc

NC = 2
NS = 16
C = 125
G = 16


def _segment_sum_sc(h, src3d, dst3d):
  n, d = h.shape
  nw, kpt, c = src3d.shape
  npad = ((n + 8 * NS - 1) // (8 * NS)) * (8 * NS)
  rpt = npad // NS
  mesh = plsc.VectorSubcoreMesh(core_axis_name="c", subcore_axis_name="s")

  @functools.partial(
      pl.kernel,
      out_type=jax.ShapeDtypeStruct((NC * npad, d), jnp.float32),
      mesh=mesh,
      scratch_types=[
          pltpu.VMEM_SHARED((npad, d), jnp.float32),
          pltpu.VMEM((G, c), jnp.int32),
          pltpu.VMEM((G, c), jnp.int32),
          pltpu.VMEM((G, c), jnp.int32),
          pltpu.VMEM((G, c), jnp.int32),
          pltpu.VMEM((c, d), jnp.float32),
          pltpu.VMEM((c, d), jnp.float32),
          pltpu.SemaphoreType.DMA,
          pltpu.SemaphoreType.DMA,
          pltpu.SemaphoreType.DMA,
          pltpu.SemaphoreType.DMA,
      ],
  )
  def seg_sum(h_hbm, src_hbm, dst_hbm, out_hbm, acc,
              srcg0, dstg0, srcg1, dstg1, rows0, rows1,
              sem0, sem1, semi0, semi1):
    cid = lax.axis_index("c")
    sid = lax.axis_index("s")
    wid = cid * NS + sid
    ngroups = kpt // G
    npairs = G // 2
    srcg = (srcg0, srcg1)
    dstg = (dstg0, dstg1)
    semi = (semi0, semi1)

    def prefetch_idx(g):
      pltpu.async_copy(src_hbm.at[wid, pl.ds(g * G, G)], srcg[g % 2],
                       semi[g % 2])
      pltpu.async_copy(dst_hbm.at[wid, pl.ds(g * G, G)], dstg[g % 2],
                       semi[g % 2])

    def wait_idx(g):
      pltpu.make_async_copy(src_hbm.at[wid, pl.ds(g * G, G)], srcg[g % 2],
                            semi[g % 2]).wait()
      pltpu.make_async_copy(dst_hbm.at[wid, pl.ds(g * G, G)], dstg[g % 2],
                            semi[g % 2]).wait()

    prefetch_idx(0)
    if ngroups > 1:
      prefetch_idx(1)
    zv = jnp.zeros((16,), jnp.float32)

    def zrow(r, carry):
      for j in range(d // 16):
        rows0[r, pl.ds(j * 16, 16)] = zv
      return carry

    lax.fori_loop(0, c, zrow, 0)
    rz = sid * rpt
    zc = (c // 8) * 8
    for t in range(rpt // zc):
      pltpu.sync_copy(rows0.at[pl.ds(0, zc)],
                      acc.at[pl.ds(rz + t * zc, zc)])
    if rpt % zc:
      pltpu.sync_copy(rows0.at[pl.ds(0, rpt % zc)],
                      acc.at[pl.ds(rz + (rpt // zc) * zc, rpt % zc)])
    plsc.subcore_barrier()
    wait_idx(0)
    pltpu.async_copy(h_hbm.at[srcg0.at[0]], rows0, sem0)

    for g in range(ngroups):
      sg, dg = srcg[g % 2], dstg[g % 2]
      if g + 1 < ngroups:
        wait_idx(g + 1)

      def pair(j, carry, sg=sg, dg=dg):
        k = j * 2
        pltpu.make_async_copy(h_hbm.at[sg.at[k]], rows0, sem0).wait()
        d1 = pltpu.async_copy(h_hbm.at[sg.at[k + 1]], rows1, sem1)
        pltpu.sync_copy(rows0, acc.at[dg.at[k]], add=True)
        pltpu.async_copy(h_hbm.at[sg.at[k + 2]], rows0, sem0)
        d1.wait()
        pltpu.sync_copy(rows1, acc.at[dg.at[k + 1]], add=True)
        return carry

      lax.fori_loop(0, npairs - 1, pair, 0)
      k = G - 2
      pltpu.make_async_copy(h_hbm.at[sg.at[k]], rows0, sem0).wait()
      d1 = pltpu.async_copy(h_hbm.at[sg.at[k + 1]], rows1, sem1)
      pltpu.sync_copy(rows0, acc.at[dg.at[k]], add=True)
      if g + 1 < ngroups:
        pltpu.async_copy(h_hbm.at[srcg[(g + 1) % 2].at[0]], rows0, sem0)
      d1.wait()
      pltpu.sync_copy(rows1, acc.at[dg.at[k + 1]], add=True)
      if g + 2 < ngroups:
        prefetch_idx(g + 2)
    plsc.subcore_barrier()
    pltpu.sync_copy(acc.at[pl.ds(rz, rpt)],
                    out_hbm.at[pl.ds(cid * npad + rz, rpt)])

  return seg_sum(h, src3d, dst3d), npad


def _mm_kernel(x_ref, w_ref, o_ref):
  o_ref[...] = jnp.dot(x_ref[...], w_ref[...],
                       preferred_element_type=jnp.float32)


def _relu_mm_kernel(n, p_ref, w_ref, o_ref):
  h = jnp.maximum(p_ref[0, :n] + p_ref[1, :n], 0.0)
  o_ref[...] = jnp.dot(h, w_ref[...],
                       preferred_element_type=jnp.float32)


def _add_log_softmax_kernel(n, p_ref, o_ref):
  h = p_ref[0, :n] + p_ref[1, :n]
  m = jnp.max(h, axis=1, keepdims=True)
  lse = jnp.log(jnp.sum(jnp.exp(h - m), axis=1, keepdims=True)) + m
  o_ref[...] = h - lse


def kernel(x, edge_index, W1, W2):
  n, d = x.shape
  e = edge_index.shape[1]
  nw = NC * NS
  kpt = e // (C * nw)
  npad = ((n + 8 * NS - 1) // (8 * NS)) * (8 * NS)
  dst3d = edge_index[0].reshape(nw, kpt, C)
  src3d = edge_index[1].reshape(nw, kpt, C)

  h1 = pl.pallas_call(
      _mm_kernel,
      out_shape=jax.ShapeDtypeStruct((n, d), jnp.float32),
  )(x, W1)
  p1, _ = _segment_sum_sc(h1, src3d, dst3d)
  h2 = pl.pallas_call(
      functools.partial(_relu_mm_kernel, n),
      out_shape=jax.ShapeDtypeStruct((n, d), jnp.float32),
  )(p1.reshape(NC, npad, d), W2)
  p2, _ = _segment_sum_sc(h2, src3d, dst3d)
  return pl.pallas_call(
      functools.partial(_add_log_softmax_kernel, n),
      out_shape=jax.ShapeDtypeStruct((n, d), jnp.float32),
  )(p2.reshape(NC, npad, d))
